# baseline (speedup 1.0000x reference)
import functools

import jax
import jax.numpy as jnp
from jax import lax
from jax.experimental import pallas as pl
from jax.experimental.pallas import tpu as pltpu
from jax.experimental.pallas import tpu_sc as plsc

NUM_CORES = 2
NUM_SUBCORES = 16
NUM_WORKERS = NUM_CORES * NUM_SUBCORES
LANES = 16

CHUNK_ROWS = 8
NBUF = 4
NPE = 2
LOOKAHEAD = 3
ADD_UNROLL = 8


@functools.partial(jax.jit, static_argnames=("batch", "seq", "d"))
def _sc_broadcast_add(x_flat, pe_flat, *, batch, seq, d):
    rows_per_w = seq // NUM_WORKERS
    chunk = CHUNK_ROWS * d
    n_chunks = rows_per_w // CHUNK_ROWS
    n_steps = n_chunks * batch
    seq_elems = seq * d

    mesh = plsc.VectorSubcoreMesh(core_axis_name="c", subcore_axis_name="s")

    scratch = (
        [pltpu.VMEM((chunk,), jnp.float32) for _ in range(NBUF + NPE)]
        + [pltpu.SemaphoreType.DMA for _ in range(2 * NBUF + NPE)]
    )

    @functools.partial(
        pl.kernel,
        out_type=jax.ShapeDtypeStruct((batch * seq * d,), jnp.float32),
        mesh=mesh,
        scratch_types=scratch,
    )
    def body(x_hbm, pe_hbm, out_hbm, *refs):
        x_bufs = refs[:NBUF]
        pe_bufs = refs[NBUF:NBUF + NPE]
        sems = refs[NBUF + NPE:]
        x_sems = sems[:NBUF]
        o_sems = sems[NBUF:2 * NBUF]
        pe_sems = sems[2 * NBUF:]

        cid = lax.axis_index("c")
        sid = lax.axis_index("s")
        wid = sid * NUM_CORES + cid
        base = wid * rows_per_w * d

        pe_descs = [None] * n_chunks
        x_descs = [None] * n_steps
        o_descs = [None] * n_steps

        def issue_loads(s):
            c, b = divmod(s, batch)
            off = base + c * chunk
            if b == 0:
                pe_descs[c] = pltpu.async_copy(
                    pe_hbm.at[pl.ds(off, chunk)], pe_bufs[c % NPE],
                    pe_sems[c % NPE])
            if s >= NBUF:
                o_descs[s - NBUF].wait()
            x_descs[s] = pltpu.async_copy(
                x_hbm.at[pl.ds(b * seq_elems + off, chunk)],
                x_bufs[s % NBUF], x_sems[s % NBUF])

        def process(s):
            c, b = divmod(s, batch)
            off = base + c * chunk
            if b == 0:
                pe_descs[c].wait()
            x_descs[s].wait()
            x_buf = x_bufs[s % NBUF]
            pe_buf = pe_bufs[c % NPE]

            @plsc.parallel_loop(0, chunk, step=LANES * ADD_UNROLL)
            def add_iter(i):
                for j in range(ADD_UNROLL):
                    sl = pl.ds(i + j * LANES, LANES)
                    x_buf[sl] = x_buf[sl] + pe_buf[sl]

            o_descs[s] = pltpu.async_copy(
                x_buf, out_hbm.at[pl.ds(b * seq_elems + off, chunk)],
                o_sems[s % NBUF])

        for s in range(LOOKAHEAD):
            issue_loads(s)
        for s in range(n_steps):
            if s + LOOKAHEAD < n_steps:
                issue_loads(s + LOOKAHEAD)
            process(s)
        for s in range(n_steps - NBUF, n_steps):
            o_descs[s].wait()

    return body(x_flat, pe_flat)


def kernel(x, pos_embedding):
    batch, seq, d = x.shape
    pe = pos_embedding[:seq]
    out_flat = _sc_broadcast_add(
        x.reshape(-1), pe.reshape(-1), batch=batch, seq=seq, d=d
    )
    return out_flat.reshape(batch, seq, d)


# speedup vs baseline: 2.3984x; 2.3984x over previous
import functools

import jax
import jax.numpy as jnp
from jax import lax
from jax.experimental import pallas as pl
from jax.experimental.pallas import tpu as pltpu
from jax.experimental.pallas import tpu_sc as plsc

NUM_CORES = 2
NUM_SUBCORES = 16
NUM_WORKERS = NUM_CORES * NUM_SUBCORES
LANES = 16

CHUNK_ROWS = 8
NBUF = 4
NPE = 2
LOOKAHEAD = 3
ADD_UNROLL = 8


@functools.partial(jax.jit, static_argnames=("batch", "seq", "d"))
def _sc_broadcast_add(x_rows, pe, *, batch, seq, d):
    rows_per_w = seq // NUM_WORKERS
    n_chunks = rows_per_w // CHUNK_ROWS
    n_steps = n_chunks * batch

    mesh = plsc.VectorSubcoreMesh(core_axis_name="c", subcore_axis_name="s")

    scratch = (
        [pltpu.VMEM((CHUNK_ROWS, d), jnp.float32) for _ in range(NBUF + NPE)]
        + [pltpu.SemaphoreType.DMA for _ in range(2 * NBUF + NPE)]
    )

    @functools.partial(
        pl.kernel,
        out_type=jax.ShapeDtypeStruct((batch * seq, d), jnp.float32),
        mesh=mesh,
        scratch_types=scratch,
    )
    def body(x_hbm, pe_hbm, out_hbm, *refs):
        x_bufs = refs[:NBUF]
        pe_bufs = refs[NBUF:NBUF + NPE]
        sems = refs[NBUF + NPE:]
        x_sems = sems[:NBUF]
        o_sems = sems[NBUF:2 * NBUF]
        pe_sems = sems[2 * NBUF:]

        cid = lax.axis_index("c")
        sid = lax.axis_index("s")
        wid = sid * NUM_CORES + cid
        base_row = wid * rows_per_w

        pe_descs = [None] * n_chunks
        x_descs = [None] * n_steps
        o_descs = [None] * n_steps

        def issue_loads(s):
            c, b = divmod(s, batch)
            row = base_row + c * CHUNK_ROWS
            if b == 0:
                pe_descs[c] = pltpu.async_copy(
                    pe_hbm.at[pl.ds(row, CHUNK_ROWS), :], pe_bufs[c % NPE],
                    pe_sems[c % NPE])
            if s >= NBUF:
                o_descs[s - NBUF].wait()
            x_descs[s] = pltpu.async_copy(
                x_hbm.at[pl.ds(b * seq + row, CHUNK_ROWS), :],
                x_bufs[s % NBUF], x_sems[s % NBUF])

        def process(s):
            c, b = divmod(s, batch)
            row = base_row + c * CHUNK_ROWS
            if b == 0:
                pe_descs[c].wait()
            x_descs[s].wait()
            x_buf = x_bufs[s % NBUF]
            pe_buf = pe_bufs[c % NPE]

            def row_body(r, _):
                @plsc.parallel_loop(0, d, step=LANES * ADD_UNROLL)
                def add_iter(i):
                    for j in range(ADD_UNROLL):
                        sl = pl.ds(i + j * LANES, LANES)
                        x_buf[r, sl] = x_buf[r, sl] + pe_buf[r, sl]
                return 0

            lax.fori_loop(0, CHUNK_ROWS, row_body, 0)

            o_descs[s] = pltpu.async_copy(
                x_buf, out_hbm.at[pl.ds(b * seq + row, CHUNK_ROWS), :],
                o_sems[s % NBUF])

        for s in range(LOOKAHEAD):
            issue_loads(s)
        for s in range(n_steps):
            if s + LOOKAHEAD < n_steps:
                issue_loads(s + LOOKAHEAD)
            process(s)
        for s in range(n_steps - NBUF, n_steps):
            o_descs[s].wait()

    return body(x_rows, pe)


def kernel(x, pos_embedding):
    batch, seq, d = x.shape
    pe = pos_embedding[:seq]
    out_rows = _sc_broadcast_add(
        x.reshape(batch * seq, d), pe, batch=batch, seq=seq, d=d
    )
    return out_rows.reshape(batch, seq, d)


# speedup vs baseline: 2.7796x; 1.1589x over previous
import functools

import jax
import jax.numpy as jnp
from jax import lax
from jax.experimental import pallas as pl
from jax.experimental.pallas import tpu as pltpu
from jax.experimental.pallas import tpu_sc as plsc

NUM_CORES = 2
NUM_SUBCORES = 16
NUM_WORKERS = NUM_CORES * NUM_SUBCORES
LANES = 16

CHUNK_ROWS = 8
NBUF = 4
NPE = 2
LOOKAHEAD = 3
ADD_UNROLL = 8

SC_ROWS = 512
TC_BS = 256


def _sc_stage(x_rows, pe, *, batch, seq, d):
    rows_per_w = SC_ROWS // NUM_WORKERS
    n_chunks = rows_per_w // CHUNK_ROWS
    n_steps = n_chunks * batch

    mesh = plsc.VectorSubcoreMesh(core_axis_name="c", subcore_axis_name="s")

    scratch = (
        [pltpu.VMEM((CHUNK_ROWS, d), jnp.float32) for _ in range(NBUF + NPE)]
        + [pltpu.SemaphoreType.DMA for _ in range(2 * NBUF + NPE)]
    )

    @functools.partial(
        pl.kernel,
        out_type=jax.ShapeDtypeStruct((batch * seq, d), jnp.float32),
        mesh=mesh,
        scratch_types=scratch,
    )
    def body(x_hbm, pe_hbm, out_hbm, *refs):
        x_bufs = refs[:NBUF]
        pe_bufs = refs[NBUF:NBUF + NPE]
        sems = refs[NBUF + NPE:]
        x_sems = sems[:NBUF]
        o_sems = sems[NBUF:2 * NBUF]
        pe_sems = sems[2 * NBUF:]

        cid = lax.axis_index("c")
        sid = lax.axis_index("s")
        wid = sid * NUM_CORES + cid
        base_row = wid * rows_per_w

        pe_descs = [None] * n_chunks
        x_descs = [None] * n_steps
        o_descs = [None] * n_steps

        def issue_loads(s):
            c, b = divmod(s, batch)
            row = base_row + c * CHUNK_ROWS
            if b == 0:
                pe_descs[c] = pltpu.async_copy(
                    pe_hbm.at[pl.ds(row, CHUNK_ROWS), :], pe_bufs[c % NPE],
                    pe_sems[c % NPE])
            if s >= NBUF:
                o_descs[s - NBUF].wait()
            x_descs[s] = pltpu.async_copy(
                x_hbm.at[pl.ds(b * seq + row, CHUNK_ROWS), :],
                x_bufs[s % NBUF], x_sems[s % NBUF])

        def process(s):
            c, b = divmod(s, batch)
            row = base_row + c * CHUNK_ROWS
            if b == 0:
                pe_descs[c].wait()
            x_descs[s].wait()
            x_buf = x_bufs[s % NBUF]
            pe_buf = pe_bufs[c % NPE]

            def row_body(r, _):
                @plsc.parallel_loop(0, d, step=LANES * ADD_UNROLL)
                def add_iter(i):
                    for j in range(ADD_UNROLL):
                        sl = pl.ds(i + j * LANES, LANES)
                        x_buf[r, sl] = x_buf[r, sl] + pe_buf[r, sl]
                return 0

            lax.fori_loop(0, CHUNK_ROWS, row_body, 0)

            o_descs[s] = pltpu.async_copy(
                x_buf, out_hbm.at[pl.ds(b * seq + row, CHUNK_ROWS), :],
                o_sems[s % NBUF])

        for s in range(LOOKAHEAD):
            issue_loads(s)
        for s in range(n_steps):
            if s + LOOKAHEAD < n_steps:
                issue_loads(s + LOOKAHEAD)
            process(s)
        for s in range(n_steps - NBUF, n_steps):
            o_descs[s].wait()

    return body(x_rows, pe)


def _tc_stage(x, pe, sc_out, *, batch, seq, d):
    n_blocks = (seq - SC_ROWS) // TC_BS
    blk0 = SC_ROWS // TC_BS

    def body(x_ref, pe_ref, alias_ref, o_ref):
        del alias_ref
        o_ref[...] = x_ref[...] + pe_ref[...]

    return pl.pallas_call(
        body,
        grid=(n_blocks, batch),
        in_specs=[
            pl.BlockSpec((1, TC_BS, d), lambda i, b: (b, i + blk0, 0)),
            pl.BlockSpec((TC_BS, d), lambda i, b: (i + blk0, 0)),
            pl.BlockSpec(memory_space=pl.ANY),
        ],
        out_specs=pl.BlockSpec((1, TC_BS, d), lambda i, b: (b, i + blk0, 0)),
        out_shape=jax.ShapeDtypeStruct((batch, seq, d), jnp.float32),
        input_output_aliases={2: 0},
        compiler_params=pltpu.CompilerParams(
            dimension_semantics=("arbitrary", "arbitrary"),
        ),
    )(x, pe, sc_out)


@functools.partial(jax.jit, static_argnames=("batch", "seq", "d"))
def _hybrid_broadcast_add(x, pe, *, batch, seq, d):
    sc_out = _sc_stage(x.reshape(batch * seq, d), pe,
                       batch=batch, seq=seq, d=d)
    return _tc_stage(x, pe, sc_out.reshape(batch, seq, d),
                     batch=batch, seq=seq, d=d)


def kernel(x, pos_embedding):
    batch, seq, d = x.shape
    pe = pos_embedding[:seq]
    return _hybrid_broadcast_add(x, pe, batch=batch, seq=seq, d=d)


# speedup vs baseline: 2.8527x; 1.0263x over previous
import functools

import jax
import jax.numpy as jnp
from jax import lax
from jax.experimental import pallas as pl
from jax.experimental.pallas import tpu as pltpu
from jax.experimental.pallas import tpu_sc as plsc

NUM_CORES = 2
NUM_SUBCORES = 16
NUM_WORKERS = NUM_CORES * NUM_SUBCORES
LANES = 16

CHUNK_ROWS = 8
NBUF = 4
NPE = 2
LOOKAHEAD = 3
ADD_UNROLL = 8

SC_ROWS = 256
TC_BS = 256


def _sc_stage(x_rows, pe, *, batch, seq, d):
    rows_per_w = SC_ROWS // NUM_WORKERS
    n_chunks = rows_per_w // CHUNK_ROWS
    n_steps = n_chunks * batch

    mesh = plsc.VectorSubcoreMesh(core_axis_name="c", subcore_axis_name="s")

    scratch = (
        [pltpu.VMEM((CHUNK_ROWS, d), jnp.float32) for _ in range(NBUF + NPE)]
        + [pltpu.SemaphoreType.DMA for _ in range(2 * NBUF + NPE)]
    )

    @functools.partial(
        pl.kernel,
        out_type=jax.ShapeDtypeStruct((batch * seq, d), jnp.float32),
        mesh=mesh,
        scratch_types=scratch,
    )
    def body(x_hbm, pe_hbm, out_hbm, *refs):
        x_bufs = refs[:NBUF]
        pe_bufs = refs[NBUF:NBUF + NPE]
        sems = refs[NBUF + NPE:]
        x_sems = sems[:NBUF]
        o_sems = sems[NBUF:2 * NBUF]
        pe_sems = sems[2 * NBUF:]

        cid = lax.axis_index("c")
        sid = lax.axis_index("s")
        wid = sid * NUM_CORES + cid
        base_row = wid * rows_per_w

        pe_descs = [None] * n_chunks
        x_descs = [None] * n_steps
        o_descs = [None] * n_steps

        def issue_loads(s):
            c, b = divmod(s, batch)
            row = base_row + c * CHUNK_ROWS
            if b == 0:
                pe_descs[c] = pltpu.async_copy(
                    pe_hbm.at[pl.ds(row, CHUNK_ROWS), :], pe_bufs[c % NPE],
                    pe_sems[c % NPE])
            if s >= NBUF:
                o_descs[s - NBUF].wait()
            x_descs[s] = pltpu.async_copy(
                x_hbm.at[pl.ds(b * seq + row, CHUNK_ROWS), :],
                x_bufs[s % NBUF], x_sems[s % NBUF])

        def process(s):
            c, b = divmod(s, batch)
            row = base_row + c * CHUNK_ROWS
            if b == 0:
                pe_descs[c].wait()
            x_descs[s].wait()
            x_buf = x_bufs[s % NBUF]
            pe_buf = pe_bufs[c % NPE]

            def row_body(r, _):
                @plsc.parallel_loop(0, d, step=LANES * ADD_UNROLL)
                def add_iter(i):
                    for j in range(ADD_UNROLL):
                        sl = pl.ds(i + j * LANES, LANES)
                        x_buf[r, sl] = x_buf[r, sl] + pe_buf[r, sl]
                return 0

            lax.fori_loop(0, CHUNK_ROWS, row_body, 0)

            o_descs[s] = pltpu.async_copy(
                x_buf, out_hbm.at[pl.ds(b * seq + row, CHUNK_ROWS), :],
                o_sems[s % NBUF])

        for s in range(LOOKAHEAD):
            issue_loads(s)
        for s in range(n_steps):
            if s + LOOKAHEAD < n_steps:
                issue_loads(s + LOOKAHEAD)
            process(s)
        for s in range(n_steps - NBUF, n_steps):
            o_descs[s].wait()

    return body(x_rows, pe)


def _tc_stage(x, pe, sc_out, *, batch, seq, d):
    n_blocks = (seq - SC_ROWS) // TC_BS
    blk0 = SC_ROWS // TC_BS

    def body(x_ref, pe_ref, alias_ref, o_ref):
        del alias_ref
        o_ref[...] = x_ref[...] + pe_ref[...]

    return pl.pallas_call(
        body,
        grid=(n_blocks, batch),
        in_specs=[
            pl.BlockSpec((1, TC_BS, d), lambda i, b: (b, i + blk0, 0)),
            pl.BlockSpec((TC_BS, d), lambda i, b: (i + blk0, 0)),
            pl.BlockSpec(memory_space=pl.ANY),
        ],
        out_specs=pl.BlockSpec((1, TC_BS, d), lambda i, b: (b, i + blk0, 0)),
        out_shape=jax.ShapeDtypeStruct((batch, seq, d), jnp.float32),
        input_output_aliases={2: 0},
        compiler_params=pltpu.CompilerParams(
            dimension_semantics=("arbitrary", "arbitrary"),
        ),
    )(x, pe, sc_out)


@functools.partial(jax.jit, static_argnames=("batch", "seq", "d"))
def _hybrid_broadcast_add(x, pe, *, batch, seq, d):
    sc_out = _sc_stage(x.reshape(batch * seq, d), pe,
                       batch=batch, seq=seq, d=d)
    return _tc_stage(x, pe, sc_out.reshape(batch, seq, d),
                     batch=batch, seq=seq, d=d)


def kernel(x, pos_embedding):
    batch, seq, d = x.shape
    pe = pos_embedding[:seq]
    return _hybrid_broadcast_add(x, pe, batch=batch, seq=seq, d=d)
